# Initial kernel scaffold; baseline (speedup 1.0000x reference)
import jax
import jax.numpy as jnp
from jax.experimental import pallas as pl


def kernel(x, edge_index, edge_attr, batch, conv1_w, conv1_b, gcn_w, gcn_b, conv2_w, conv2_b):
    raise NotImplementedError("write your pallas kernel here")


# speedup vs baseline: 30.2947x; 30.2947x over previous
import jax
import jax.numpy as jnp
from jax import lax
from jax.experimental import pallas as pl
from jax.experimental.pallas import tpu as pltpu
from jax.experimental.pallas import tpu_sc as plsc

N = 10000
NPAD = 10240
E = 320000
T = 12
TP1 = 10
TP2 = 8
CIN = 128
HID = 128
GCN_OUT = 32
BN = 1000

NTILES = 16
EBLK = 128
BLKS_PER_TILE = 158
HBLK = BLKS_PER_TILE // 2
EPT = BLKS_PER_TILE * EBLK
EPAD = NTILES * EPT
EBLK2 = 64
NBLK2 = EPT // EBLK2
HALF = 160
DROWS = NPAD // 16


def _ka_body(xt_ref, w1_ref, b1_ref, gw_ref, lo_ref, hi_ref):
    b1 = b1_ref[...]
    gw = gw_ref[...]
    for t in range(TP1):
        y = b1
        for k in range(3):
            y = y + lax.dot_general(
                xt_ref[:, t + k, :], w1_ref[k],
                (((1,), (0,)), ((), ())),
                preferred_element_type=jnp.float32)
        h = y[:, :HID] * jax.nn.sigmoid(y[:, HID:])
        xl = lax.dot_general(h, gw, (((1,), (0,)), ((), ())),
                             preferred_element_type=jnp.float32)
        tgt = lo_ref if t < 5 else hi_ref
        c = (t % 5) * GCN_OUT
        tgt[:, c:c + GCN_OUT] = xl


def _kernel_a(xt, w1, b1, gw):
    return pl.pallas_call(
        _ka_body,
        grid=(N // BN,),
        in_specs=[
            pl.BlockSpec((BN, T, CIN), lambda i: (i, 0, 0)),
            pl.BlockSpec((3, CIN, 2 * HID), lambda i: (0, 0, 0)),
            pl.BlockSpec((1, 2 * HID), lambda i: (0, 0)),
            pl.BlockSpec((CIN, GCN_OUT), lambda i: (0, 0)),
        ],
        out_specs=[
            pl.BlockSpec((BN, HALF), lambda i: (i, 0)),
            pl.BlockSpec((BN, HALF), lambda i: (i, 0)),
        ],
        out_shape=[
            jax.ShapeDtypeStruct((N, HALF), jnp.float32),
            jax.ShapeDtypeStruct((N, HALF), jnp.float32),
        ],
    )(xt, w1, b1, gw)


def _kb1_body(dstq_hbm, ew16_hbm, deg0_hbm, deg1_hbm,
              dstq_ref, ewrows_ref, deg_sh):
    cid = lax.axis_index("c")
    tid = lax.axis_index("s")
    zeros16f = jnp.zeros((16,), jnp.float32)

    pltpu.sync_copy(dstq_hbm.at[tid], dstq_ref)

    def zewrows(r, carry):
        ewrows_ref[r, :] = zeros16f
        return carry
    lax.fori_loop(0, 40, zewrows, 0)
    pltpu.sync_copy(ewrows_ref.at[pl.ds(0, 40)],
                    deg_sh.at[pl.ds(tid * 40, 40)])
    plsc.subcore_barrier()

    def deg_blk(j, carry):
        pltpu.sync_copy(ew16_hbm.at[tid, j], ewrows_ref)
        pltpu.sync_copy(ewrows_ref, deg_sh.at[dstq_ref.at[j]], add=True)
        return carry
    lax.fori_loop(cid * HBLK, (cid + 1) * HBLK, deg_blk, 0)
    plsc.subcore_barrier()

    sl = pl.ds(tid * 40, 40)
    pltpu.sync_copy(deg_sh.at[sl], ewrows_ref.at[pl.ds(0, 40)])

    @pl.when(cid == 0)
    def _():
        pltpu.sync_copy(ewrows_ref.at[pl.ds(0, 40)], deg0_hbm.at[sl])

    @pl.when(cid == 1)
    def _():
        pltpu.sync_copy(ewrows_ref.at[pl.ds(0, 40)], deg1_hbm.at[sl])


def _kernel_b1(dstq, ew16):
    mesh = plsc.VectorSubcoreMesh(core_axis_name="c", subcore_axis_name="s")
    return pl.kernel(
        _kb1_body,
        out_type=[
            jax.ShapeDtypeStruct((DROWS, 16), jnp.float32),
            jax.ShapeDtypeStruct((DROWS, 16), jnp.float32),
        ],
        mesh=mesh,
        compiler_params=pltpu.CompilerParams(needs_layout_passes=False,
                                             use_tc_tiling_on_sc=False),
        scratch_types=[
            pltpu.VMEM((BLKS_PER_TILE, EBLK), jnp.int32),
            pltpu.VMEM((EBLK, 16), jnp.float32),
            pltpu.VMEM_SHARED((DROWS, 16), jnp.float32),
        ],
    )(dstq, ew16)


def _kd_body(d0_ref, d1_ref, dinv_ref):
    dinv_ref[...] = lax.rsqrt(d0_ref[...] + d1_ref[...] + 1.0)


def _kernel_d(deg0, deg1):
    return pl.pallas_call(
        _kd_body,
        out_shape=jax.ShapeDtypeStruct((DROWS, 16), jnp.float32),
    )(deg0, deg1)


def _kb2_body(src_hbm, dst_hbm, ew_hbm, dinv_hbm, xl_hbm, acc_hbm,
              srcb, dstb, ewb, rows_ref, norm_ref, dinv_ref,
              acc_sh, sem):
    cid = lax.axis_index("c")
    tid = lax.axis_index("s")
    zeros16f = jnp.zeros((16,), jnp.float32)

    pltpu.sync_copy(dinv_hbm, dinv_ref)

    def zrows(r, carry):
        for jj in range(10):
            rows_ref[r, pl.ds(jj * 16, 16)] = zeros16f
        return carry
    lax.fori_loop(0, EBLK2, zrows, 0)
    for i in range(10):
        pltpu.sync_copy(rows_ref, acc_sh.at[pl.ds(tid * 640 + i * EBLK2, EBLK2)])
    plsc.subcore_barrier()

    goff = cid * N

    def edge_blk(j, carry):
        pltpu.sync_copy(src_hbm.at[tid, j], srcb)
        pltpu.sync_copy(dst_hbm.at[tid, j], dstb)
        pltpu.sync_copy(ew_hbm.at[tid, j], ewb)
        for g in range(EBLK2 // 16):
            sl = pl.ds(g * 16, 16)
            s16 = srcb[0, sl]
            d16 = dstb[0, sl]
            nrm = (plsc.load_gather(dinv_ref, [s16]) * ewb[sl]
                   * plsc.load_gather(dinv_ref, [d16]))
            norm_ref[sl] = nrm
            srcb[0, sl] = s16 + goff
        pltpu.async_copy(xl_hbm.at[srcb.at[0]], rows_ref, sem).wait()

        def scale(g, c2):
            nv = norm_ref[pl.ds(g * 16, 16)]
            for lane in range(16):
                s = nv[lane]
                base = g * 16 + lane
                for jj in range(10):
                    sl2 = pl.ds(jj * 16, 16)
                    rows_ref[base, sl2] = rows_ref[base, sl2] * s
            return c2
        lax.fori_loop(0, EBLK2 // 16, scale, 0)
        pltpu.sync_copy(rows_ref, acc_sh.at[dstb.at[0]], add=True)
        return carry
    lax.fori_loop(0, NBLK2, edge_blk, 0)
    plsc.subcore_barrier()

    for i in range(10):
        sl = pl.ds(tid * 640 + i * EBLK2, EBLK2)
        pltpu.sync_copy(acc_sh.at[sl], rows_ref)
        pltpu.sync_copy(rows_ref,
                        acc_hbm.at[pl.ds(cid * NPAD + tid * 640 + i * EBLK2, EBLK2)])


def _kernel_b2(srcp, dstp, ewp, dinv, xl_cat):
    mesh = plsc.VectorSubcoreMesh(core_axis_name="c", subcore_axis_name="s")
    return pl.kernel(
        _kb2_body,
        out_type=[
            jax.ShapeDtypeStruct((2 * NPAD, HALF), jnp.float32),
        ],
        mesh=mesh,
        compiler_params=pltpu.CompilerParams(needs_layout_passes=False,
                                             use_tc_tiling_on_sc=False),
        scratch_types=[
            pltpu.VMEM((1, EBLK2), jnp.int32),
            pltpu.VMEM((1, EBLK2), jnp.int32),
            pltpu.VMEM((EBLK2,), jnp.float32),
            pltpu.VMEM((EBLK2, HALF), jnp.float32),
            pltpu.VMEM((EBLK2,), jnp.float32),
            pltpu.VMEM((NPAD,), jnp.float32),
            pltpu.VMEM_SHARED((NPAD, HALF), jnp.float32),
            pltpu.SemaphoreType.DMA,
        ],
    )(srcp, dstp, ewp, dinv, xl_cat)


def _kc_body(acc_ref, xl_ref, dinv_ref, gb_ref, w2_ref, b2_ref, out_ref):
    dv = dinv_ref[...]
    g = jnp.maximum(acc_ref[...] + dv * dv * xl_ref[...] + gb_ref[...], 0.0)
    z = lax.dot_general(g, w2_ref[...], (((1,), (0,)), ((), ())),
                        preferred_element_type=jnp.float32)
    b2 = b2_ref[...]
    for t in range(TP2):
        zt = z[:, t * 256:(t + 1) * 256] + b2
        out_ref[t] = zt[:, :HID] * jax.nn.sigmoid(zt[:, HID:])


def _kernel_c(acc320, xl320, dinv2, gb320, w2big, b2):
    return pl.pallas_call(
        _kc_body,
        grid=(N // BN,),
        in_specs=[
            pl.BlockSpec((BN, 2 * HALF), lambda i: (i, 0)),
            pl.BlockSpec((BN, 2 * HALF), lambda i: (i, 0)),
            pl.BlockSpec((BN, 1), lambda i: (i, 0)),
            pl.BlockSpec((1, 2 * HALF), lambda i: (0, 0)),
            pl.BlockSpec((2 * HALF, TP2 * 256), lambda i: (0, 0)),
            pl.BlockSpec((1, 256), lambda i: (0, 0)),
        ],
        out_specs=pl.BlockSpec((TP2, BN, HID), lambda i: (0, i, 0)),
        out_shape=jax.ShapeDtypeStruct((TP2, N, HID), jnp.float32),
    )(acc320, xl320, dinv2, gb320, w2big, b2)


@jax.jit
def kernel(x, edge_index, edge_attr, batch, conv1_w, conv1_b, gcn_w, gcn_b,
           conv2_w, conv2_b):
    del batch
    xt = jnp.transpose(x, (0, 2, 1))
    w1 = jnp.transpose(conv1_w, (2, 1, 0))
    b1 = conv1_b[None, :]
    xl_lo, xl_hi = _kernel_a(xt, w1, b1, gcn_w)

    pad = EPAD - E
    fill = jnp.arange(pad, dtype=jnp.int32) % N
    src_f = jnp.concatenate([edge_index[0], fill])
    dst_f = jnp.concatenate([edge_index[1], fill])
    ew_f = jnp.concatenate([edge_attr, jnp.zeros((pad,), jnp.float32)])
    srcp = src_f.reshape(NTILES, NBLK2, 1, EBLK2)
    dstp = dst_f.reshape(NTILES, NBLK2, 1, EBLK2)
    dstq = (dst_f >> 4).reshape(NTILES, BLKS_PER_TILE, EBLK)
    ewp = ew_f.reshape(NTILES, NBLK2, EBLK2)
    ew16 = (ew_f[:, None] * jax.nn.one_hot(dst_f & 15, 16, dtype=jnp.float32)
            ).reshape(NTILES, BLKS_PER_TILE, EBLK, 16)

    deg0, deg1 = _kernel_b1(dstq, ew16)
    dinv = _kernel_d(deg0, deg1).reshape(NPAD)

    xl_cat = jnp.concatenate([xl_lo, xl_hi], axis=0)
    (acc_cat,) = _kernel_b2(srcp, dstp, ewp, dinv, xl_cat)

    acc320 = jnp.concatenate([acc_cat[:N], acc_cat[NPAD:NPAD + N]], axis=1)
    xl320 = jnp.concatenate([xl_lo, xl_hi], axis=1)
    gb320 = jnp.tile(gcn_b, TP1)[None, :]

    w2t = jnp.transpose(conv2_w, (2, 1, 0))
    w2big = jnp.zeros((2 * HALF, TP2 * 256), jnp.float32)
    for t in range(TP2):
        for k in range(3):
            w2big = w2big.at[(t + k) * GCN_OUT:(t + k + 1) * GCN_OUT,
                             t * 256:(t + 1) * 256].set(w2t[k])

    out8 = _kernel_c(acc320, xl320, dinv[:N, None], gb320, w2big,
                     conv2_b[None, :])
    return jnp.transpose(out8, (1, 2, 0))
